# baseline (speedup 1.0000x reference)
import jax
import jax.numpy as jnp
from jax.experimental import pallas as pl

MAXLEN = 3
EMBED_DIM = 640
ROW = MAXLEN * EMBED_DIM


def _add_kernel(x_ref, pos_ref, o_ref):
    o_ref[...] = x_ref[...] + pos_ref[...]


def kernel(x, pos_table):
    n = x.shape[0]
    rows = n * MAXLEN
    x2 = x.reshape(rows, EMBED_DIM)
    blk = 3072
    pos_big = jnp.tile(pos_table, (blk // MAXLEN, 1))
    out = pl.pallas_call(
        _add_kernel,
        grid=(rows // blk,),
        in_specs=[
            pl.BlockSpec((blk, EMBED_DIM), lambda i: (i, 0)),
            pl.BlockSpec((blk, EMBED_DIM), lambda i: (0, 0)),
        ],
        out_specs=pl.BlockSpec((blk, EMBED_DIM), lambda i: (i, 0)),
        out_shape=jax.ShapeDtypeStruct((rows, EMBED_DIM), x.dtype),
    )(x2, pos_big)
    return out.reshape(n, MAXLEN, EMBED_DIM)


# speedup vs baseline: 8.2849x; 8.2849x over previous
import jax
import jax.numpy as jnp
from jax.experimental import pallas as pl

MAXLEN = 3
EMBED_DIM = 640
ROW = MAXLEN * EMBED_DIM


def _add_kernel(x_ref, pos_ref, o_ref):
    o_ref[...] = x_ref[...] + pos_ref[...]


def kernel(x, pos_table):
    n = x.shape[0]
    xt = jnp.transpose(x, (1, 0, 2))
    pos3 = pos_table.reshape(MAXLEN, 1, EMBED_DIM)
    blk = 1024
    out = pl.pallas_call(
        _add_kernel,
        grid=(n // blk,),
        in_specs=[
            pl.BlockSpec((MAXLEN, blk, EMBED_DIM), lambda i: (0, i, 0)),
            pl.BlockSpec((MAXLEN, 1, EMBED_DIM), lambda i: (0, 0, 0)),
        ],
        out_specs=pl.BlockSpec((MAXLEN, blk, EMBED_DIM), lambda i: (0, i, 0)),
        out_shape=jax.ShapeDtypeStruct((MAXLEN, n, EMBED_DIM), x.dtype),
    )(xt, pos3)
    return jnp.transpose(out, (1, 0, 2))
